# baseline (speedup 1.0000x reference)
import functools

import jax
import jax.numpy as jnp
from jax import lax
from jax.experimental import pallas as pl
from jax.experimental.pallas import tpu as pltpu
from jax.experimental.pallas import tpu_sc as plsc

_NC = 2
_NS = 16
_NW = _NC * _NS
_L = 16


@functools.lru_cache(maxsize=None)
def _build(n_pil: int, n_out: int):
    per_w = n_out // _NW
    n_vec = per_w // _L
    steps = []
    s = 1
    while s * 2 < n_pil:
        s *= 2
    while s >= 1:
        steps.append(s)
        s //= 2

    mesh = plsc.VectorSubcoreMesh(
        core_axis_name="c", subcore_axis_name="s",
        num_cores=_NC, num_subcores=_NS,
    )

    @functools.partial(
        pl.kernel,
        out_type=jax.ShapeDtypeStruct((n_out,), jnp.float32),
        mesh=mesh,
        compiler_params=pltpu.CompilerParams(
            needs_layout_passes=False, skip_device_barrier=True),
        scratch_types=[
            pltpu.VMEM((n_pil,), jnp.float32),
            pltpu.VMEM((n_pil,), jnp.float32),
            pltpu.VMEM((per_w,), jnp.float32),
            pltpu.VMEM((per_w,), jnp.float32),
            pltpu.VMEM((per_w,), jnp.float32),
            pltpu.SemaphoreType.DMA,
            pltpu.SemaphoreType.DMA,
        ],
    )
    def interp(h_hbm, p_hbm, a_hbm, b_hbm, out_hbm,
               h_v, p_v, a_v, b_v, o_v, sem0, sem1):
        wid = lax.axis_index("s") * _NC + lax.axis_index("c")
        base = wid * per_w
        cp_p = pltpu.async_copy(p_hbm, p_v, sem0)
        cp_h = pltpu.async_copy(h_hbm, h_v, sem1)
        cp_a = pltpu.async_copy(a_hbm.at[pl.ds(base, per_w)], a_v, sem1)
        cp_b = pltpu.async_copy(b_hbm.at[pl.ds(base, per_w)], b_v, sem1)
        cp_p.wait()

        last = n_pil - 1
        zero = jnp.zeros((_L,), jnp.int32)
        iota = lax.iota(jnp.int32, _L)
        tfs = [(base + j * _L + iota).astype(jnp.float32)
               for j in range(n_vec)]
        def sbody(i, ps):
            step = jnp.int32(steps[0]) >> i
            out = []
            for j in range(n_vec):
                cand = jnp.minimum(ps[j] + step, last)
                pv = plsc.load_gather(p_v, [cand])
                out.append(jnp.where(pv <= tfs[j], cand, ps[j]))
            return tuple(out)

        poss = list(lax.fori_loop(0, len(steps), sbody, (zero,) * n_vec))

        cp_h.wait()
        cp_a.wait()
        cp_b.wait()

        vlast = zero + last
        h_last = plsc.load_gather(h_v, [vlast])
        h_prev = plsc.load_gather(h_v, [vlast - 1])
        p_last = plsc.load_gather(p_v, [vlast])
        p_prev = plsc.load_gather(p_v, [vlast - 1])
        slope = (h_last - h_prev) / (p_last - p_prev)
        h_ext = h_last + slope * (float(n_out - 1) - p_last)

        half = n_vec // 2
        for j in range(n_vec):
            left = poss[j]
            right = left + 1
            y_b = plsc.load_gather(h_v, [left])
            y_a = jnp.where(right > last, h_ext,
                            plsc.load_gather(h_v, [jnp.minimum(right, last)]))
            sl = pl.ds(j * _L, _L)
            o_v[sl] = a_v[sl] * y_a + b_v[sl] * y_b
            if j == half - 1:
                cp_o0 = pltpu.async_copy(
                    o_v.at[pl.ds(0, half * _L)],
                    out_hbm.at[pl.ds(base, half * _L)], sem0)

        cp_o1 = pltpu.async_copy(
            o_v.at[pl.ds(half * _L, per_w - half * _L)],
            out_hbm.at[pl.ds(base + half * _L, per_w - half * _L)], sem1)
        cp_o0.wait()
        cp_o1.wait()

    return interp


def kernel(LS_est, pilot_pos_1based, Nfft, interp_alpha, interp_beta):
    del Nfft
    n_out = interp_alpha.shape[0]
    n_pil = LS_est.shape[0]
    return _build(n_pil, n_out)(
        LS_est, pilot_pos_1based, interp_alpha, interp_beta)


# speedup vs baseline: 1.0066x; 1.0066x over previous
import functools

import jax
import jax.numpy as jnp
from jax import lax
from jax.experimental import pallas as pl
from jax.experimental.pallas import tpu as pltpu
from jax.experimental.pallas import tpu_sc as plsc

_NC = 2
_NS = 16
_NW = _NC * _NS
_L = 16


@functools.lru_cache(maxsize=None)
def _build(n_pil: int, n_out: int):
    per_w = n_out // _NW
    n_vec = per_w // _L
    steps = []
    s = 1
    while s * 2 < n_pil:
        s *= 2
    while s >= 1:
        steps.append(s)
        s //= 2

    mesh = plsc.VectorSubcoreMesh(
        core_axis_name="c", subcore_axis_name="s",
        num_cores=_NC, num_subcores=_NS,
    )

    @functools.partial(
        pl.kernel,
        out_type=jax.ShapeDtypeStruct((n_out,), jnp.float32),
        mesh=mesh,
        compiler_params=pltpu.CompilerParams(
            needs_layout_passes=False, skip_device_barrier=True),
        scratch_types=[
            pltpu.VMEM((n_pil,), jnp.float32),
            pltpu.VMEM((n_pil,), jnp.float32),
            pltpu.VMEM((per_w,), jnp.float32),
            pltpu.VMEM((per_w,), jnp.float32),
            pltpu.VMEM((per_w,), jnp.float32),
            pltpu.VMEM((per_w,), jnp.int32),
            pltpu.SemaphoreType.DMA,
            pltpu.SemaphoreType.DMA,
        ],
    )
    def interp(h_hbm, p_hbm, a_hbm, b_hbm, out_hbm,
               h_v, p_v, a_v, b_v, o_v, ps_v, sem0, sem1):
        wid = lax.axis_index("s") * _NC + lax.axis_index("c")
        base = wid * per_w
        cp_p = pltpu.async_copy(p_hbm, p_v, sem0)
        cp_h = pltpu.async_copy(h_hbm, h_v, sem1)
        cp_a = pltpu.async_copy(a_hbm.at[pl.ds(base, per_w)], a_v, sem1)
        cp_b = pltpu.async_copy(b_hbm.at[pl.ds(base, per_w)], b_v, sem1)
        cp_p.wait()

        last = n_pil - 1
        zero = jnp.zeros((_L,), jnp.int32)
        iota = lax.iota(jnp.int32, _L)
        tfs = [(base + j * _L + iota).astype(jnp.float32)
               for j in range(n_vec)]
        def sbody(i, ps):
            step = jnp.int32(steps[0]) >> i
            out = []
            for j in range(n_vec):
                cand = jnp.minimum(ps[j] + step, last)
                pv = plsc.load_gather(p_v, [cand])
                out.append(jnp.where(pv <= tfs[j], cand, ps[j]))
            return tuple(out)

        poss = list(lax.fori_loop(0, len(steps), sbody, (zero,) * n_vec))
        for j in range(n_vec):
            ps_v[pl.ds(j * _L, _L)] = poss[j]

        cp_h.wait()
        cp_a.wait()
        cp_b.wait()

        vlast = zero + last
        h_last = plsc.load_gather(h_v, [vlast])
        h_prev = plsc.load_gather(h_v, [vlast - 1])
        p_last = plsc.load_gather(p_v, [vlast])
        p_prev = plsc.load_gather(p_v, [vlast - 1])
        slope = (h_last - h_prev) / (p_last - p_prev)
        h_ext = h_last + slope * (float(n_out - 1) - p_last)

        def bbody(j, carry):
            off = j * _L
            sl = pl.ds(off, _L)
            left = ps_v[sl]
            right = left + 1
            y_b = plsc.load_gather(h_v, [left])
            y_a = jnp.where(right > last, h_ext,
                            plsc.load_gather(h_v, [jnp.minimum(right, last)]))
            o_v[sl] = a_v[sl] * y_a + b_v[sl] * y_b
            return carry

        lax.fori_loop(0, n_vec, bbody, 0)

        pltpu.sync_copy(o_v, out_hbm.at[pl.ds(base, per_w)])

    return interp


def kernel(LS_est, pilot_pos_1based, Nfft, interp_alpha, interp_beta):
    del Nfft
    n_out = interp_alpha.shape[0]
    n_pil = LS_est.shape[0]
    return _build(n_pil, n_out)(
        LS_est, pilot_pos_1based, interp_alpha, interp_beta)


# speedup vs baseline: 1.0111x; 1.0045x over previous
import functools

import jax
import jax.numpy as jnp
from jax import lax
from jax.experimental import pallas as pl
from jax.experimental.pallas import tpu as pltpu
from jax.experimental.pallas import tpu_sc as plsc

_NC = 2
_NS = 16
_NW = _NC * _NS
_L = 16


@functools.lru_cache(maxsize=None)
def _build(n_pil: int, n_out: int):
    per_w = n_out // _NW
    n_vec = per_w // _L
    steps = []
    s = 1
    while s * 2 < n_pil:
        s *= 2
    while s >= 1:
        steps.append(s)
        s //= 2

    mesh = plsc.VectorSubcoreMesh(
        core_axis_name="c", subcore_axis_name="s",
        num_cores=_NC, num_subcores=_NS,
    )

    @functools.partial(
        pl.kernel,
        out_type=jax.ShapeDtypeStruct((n_out,), jnp.float32),
        mesh=mesh,
        compiler_params=pltpu.CompilerParams(needs_layout_passes=False),
        scratch_types=[
            pltpu.VMEM((n_pil,), jnp.float32),
            pltpu.VMEM((n_pil,), jnp.float32),
            pltpu.VMEM((per_w,), jnp.float32),
            pltpu.VMEM((per_w,), jnp.float32),
            pltpu.VMEM((per_w,), jnp.float32),
            pltpu.VMEM((per_w,), jnp.int32),
            pltpu.SemaphoreType.DMA,
            pltpu.SemaphoreType.DMA,
        ],
    )
    def interp(h_hbm, p_hbm, a_hbm, b_hbm, out_hbm,
               h_v, p_v, a_v, b_v, o_v, ps_v, sem0, sem1):
        wid = lax.axis_index("s") * _NC + lax.axis_index("c")
        base = wid * per_w
        cp_p = pltpu.async_copy(p_hbm, p_v, sem0)
        cp_h = pltpu.async_copy(h_hbm, h_v, sem1)
        cp_a = pltpu.async_copy(a_hbm.at[pl.ds(base, per_w)], a_v, sem1)
        cp_b = pltpu.async_copy(b_hbm.at[pl.ds(base, per_w)], b_v, sem1)
        cp_p.wait()

        last = n_pil - 1
        zero = jnp.zeros((_L,), jnp.int32)
        iota = lax.iota(jnp.int32, _L)
        t0 = (base + iota).astype(jnp.float32)
        def sbody(i, ps):
            step = jnp.int32(steps[0]) >> i
            out = []
            for j in range(n_vec):
                cand = jnp.minimum(ps[j] + step, last)
                pv = plsc.load_gather(p_v, [cand])
                out.append(jnp.where(pv <= t0 + float(j * _L), cand, ps[j]))
            return tuple(out)

        poss = list(lax.fori_loop(0, len(steps), sbody, (zero,) * n_vec))
        for j in range(n_vec):
            ps_v[pl.ds(j * _L, _L)] = poss[j]

        cp_h.wait()
        cp_a.wait()
        cp_b.wait()

        vlast = zero + last
        h_last = plsc.load_gather(h_v, [vlast])
        h_prev = plsc.load_gather(h_v, [vlast - 1])
        p_last = plsc.load_gather(p_v, [vlast])
        p_prev = plsc.load_gather(p_v, [vlast - 1])
        slope = (h_last - h_prev) / (p_last - p_prev)
        h_ext = h_last + slope * (float(n_out - 1) - p_last)

        def bbody(j, carry):
            off = j * _L
            sl = pl.ds(off, _L)
            left = ps_v[sl]
            right = left + 1
            y_b = plsc.load_gather(h_v, [left])
            y_a = jnp.where(right > last, h_ext,
                            plsc.load_gather(h_v, [jnp.minimum(right, last)]))
            o_v[sl] = a_v[sl] * y_a + b_v[sl] * y_b
            return carry

        lax.fori_loop(0, n_vec, bbody, 0)

        pltpu.sync_copy(o_v, out_hbm.at[pl.ds(base, per_w)])

    return interp


def kernel(LS_est, pilot_pos_1based, Nfft, interp_alpha, interp_beta):
    del Nfft
    n_out = interp_alpha.shape[0]
    n_pil = LS_est.shape[0]
    return _build(n_pil, n_out)(
        LS_est, pilot_pos_1based, interp_alpha, interp_beta)
